# baseline (speedup 1.0000x reference)
import jax
import jax.numpy as jnp
from jax import lax
from jax.experimental import pallas as pl
from jax.experimental.pallas import tpu as pltpu

D_MODEL = 2048
N_EXP = 16
BT = 512
NSTEP = 8192 // BT
NBUF = 3


def _topk2(gl):
    iota = lax.broadcasted_iota(jnp.int32, (BT, N_EXP), 1)
    m0 = jnp.max(gl, axis=1, keepdims=True)
    i0 = jnp.min(jnp.where(gl == m0, iota, N_EXP), axis=1, keepdims=True)
    gl2 = jnp.where(iota == i0, -jnp.inf, gl)
    m1 = jnp.max(gl2, axis=1, keepdims=True)
    i1 = jnp.min(jnp.where(gl2 == m1, iota, N_EXP), axis=1, keepdims=True)
    e1 = jnp.exp(m1 - m0)
    s = 1.0 + e1
    return i0, i1, 1.0 / s, e1 / s


def _body(x_hbm, we_hbm, be_hbm, wg_hbm, bg_hbm,
          out_hbm, idx_hbm, w_hbm,
          wf, wgv, bgv, bev, xbufs, obufs, idxacc, wacc,
          sem_w, sem_misc, sem_fin, sx, so):
    cpx0 = pltpu.make_async_copy(x_hbm.at[pl.ds(0, BT), :], xbufs[0], sx[0])
    cpx0.start()
    cpx1 = pltpu.make_async_copy(x_hbm.at[pl.ds(BT, BT), :], xbufs[1], sx[1])
    cpx1.start()
    cpw = pltpu.make_async_copy(we_hbm, wf, sem_w)
    cpw.start()
    for src, dst in ((wg_hbm, wgv), (bg_hbm, bgv), (be_hbm, bev)):
        cp = pltpu.make_async_copy(src, dst, sem_misc)
        cp.start()
        cp.wait()
    cpw.wait()

    for i in range(NSTEP):
        jm = i % NBUF
        pltpu.make_async_copy(
            x_hbm.at[pl.ds(i * BT, BT), :], xbufs[jm], sx[jm]).wait()
        if i + 2 < NSTEP:
            jn = (i + 2) % NBUF
            pltpu.make_async_copy(
                x_hbm.at[pl.ds((i + 2) * BT, BT), :], xbufs[jn], sx[jn]).start()
        if i >= NBUF:
            pltpu.make_async_copy(
                obufs[jm], out_hbm.at[pl.ds((i - NBUF) * BT, BT), :],
                so[jm]).wait()
        xb = xbufs[jm][...]
        gl = lax.dot_general(xb, wgv[...], (((1,), (1,)), ((), ())),
                             preferred_element_type=jnp.float32)
        gl = gl + bgv[...]
        i0, i1, w0, w1 = _topk2(gl)
        iota2 = lax.broadcasted_iota(jnp.int32, (BT, 2), 1)
        idxacc[i] = jnp.where(iota2 == 0, i0, i1)
        wacc[i] = jnp.where(iota2 == 0, w0, w1)
        acc = lax.dot_general(xb, wf[...], (((1,), (1,)), ((), ())),
                              preferred_element_type=jnp.float32)
        obufs[jm][...] = acc + bev[...]
        pltpu.make_async_copy(
            obufs[jm], out_hbm.at[pl.ds(i * BT, BT), :], so[jm]).start()

    for i in range(NSTEP - NBUF, NSTEP):
        jm = i % NBUF
        pltpu.make_async_copy(
            obufs[jm], out_hbm.at[pl.ds(i * BT, BT), :], so[jm]).wait()
    cpi = pltpu.make_async_copy(idxacc, idx_hbm, sem_fin)
    cpi.start()
    cpi.wait()
    cpv = pltpu.make_async_copy(wacc, w_hbm, sem_fin)
    cpv.start()
    cpv.wait()


def kernel(x, W_exp, b_exp, W_gate, b_gate):
    n_tok = x.shape[0]
    bg2 = b_gate.reshape(1, N_EXP)
    be2 = b_exp.reshape(1, D_MODEL)

    out, idxp, wp = pl.pallas_call(
        _body,
        in_specs=[pl.BlockSpec(memory_space=pl.ANY)] * 5,
        out_specs=[pl.BlockSpec(memory_space=pl.ANY)] * 3,
        out_shape=[
            jax.ShapeDtypeStruct((n_tok, D_MODEL), jnp.float32),
            jax.ShapeDtypeStruct((NSTEP, BT, 2), jnp.int32),
            jax.ShapeDtypeStruct((NSTEP, BT, 2), jnp.float32),
        ],
        scratch_shapes=[
            pltpu.VMEM((D_MODEL, D_MODEL), jnp.float32),
            pltpu.VMEM((N_EXP, D_MODEL), jnp.float32),
            pltpu.VMEM((1, N_EXP), jnp.float32),
            pltpu.VMEM((1, D_MODEL), jnp.float32),
            [pltpu.VMEM((BT, D_MODEL), jnp.float32) for _ in range(NBUF)],
            [pltpu.VMEM((BT, D_MODEL), jnp.float32) for _ in range(NBUF)],
            pltpu.VMEM((NSTEP, BT, 2), jnp.int32),
            pltpu.VMEM((NSTEP, BT, 2), jnp.float32),
            pltpu.SemaphoreType.DMA,
            pltpu.SemaphoreType.DMA,
            pltpu.SemaphoreType.DMA,
            [pltpu.SemaphoreType.DMA for _ in range(NBUF)],
            [pltpu.SemaphoreType.DMA for _ in range(NBUF)],
        ],
    )(x, W_exp, be2, W_gate, bg2)
    return out, idxp.reshape(n_tok, 2), wp.reshape(n_tok, 2)
